# baseline (speedup 1.0000x reference)
import functools

import jax
import jax.numpy as jnp
from jax import lax
from jax.experimental import pallas as pl
from jax.experimental.pallas import tpu as pltpu
from jax.experimental.pallas import tpu_sc as plsc

_NUM_EMB = 8192
_EMB_DIM = 256
_COMMIT = 0.25
_EPS = 1e-05
_NTOK = 8192

_TM = 1024
_CK = 8192

_NW = 32
_BW = _NTOK // _NW


def _stats_body(zf_ref, std_ref):
    zf = zf_ref[...]
    mu = jnp.sum(zf, axis=0, keepdims=True) / float(_NTOK)
    dev = zf - mu
    var = jnp.sum(dev * dev, axis=0, keepdims=True) / float(_NTOK - 1)
    std_ref[...] = 1.0 / jnp.maximum(jnp.sqrt(var), _EPS)


def _argmin_body(zf_ref, std_ref, ew_ref, idx_ref, dsum_ref):
    t = pl.program_id(0)
    zn = zf_ref[...] * std_ref[...]
    z2 = jnp.sum(zn * zn, axis=1, keepdims=True)
    znb = zn.astype(jnp.bfloat16)
    best = jnp.full((_TM,), jnp.inf, dtype=jnp.float32)
    besti = jnp.zeros((_TM,), dtype=jnp.int32)
    for c in range(_NUM_EMB // _CK):
        ew_c = ew_ref[pl.ds(c * _CK, _CK), :]
        m = lax.dot_general(znb, ew_c.astype(jnp.bfloat16),
                            (((1,), (1,)), ((), ())),
                            preferred_element_type=jnp.float32)
        e2 = jnp.sum(ew_c * ew_c, axis=1)
        d = (z2 - 2.0 * m) + e2[None, :]
        lm = jnp.min(d, axis=1)
        la = jnp.argmin(d, axis=1).astype(jnp.int32) + c * _CK
        upd = lm < best
        besti = jnp.where(upd, la, besti)
        best = jnp.where(upd, lm, best)
    idx_ref[0, 0, :] = besti

    @pl.when(t == 0)
    def _():
        dsum_ref[0, 0] = 0.0

    dsum_ref[0, 0] += jnp.sum(best)


def _sc_gather_body(ew_hbm, idx_hbm, zq_hbm, hist_hbm,
                    idxv, rows_a, rows_b, hist, sem):
    wid = lax.axis_index("s") * 2 + lax.axis_index("c")
    base = wid * _BW
    pltpu.sync_copy(idx_hbm.at[pl.ds(base, _BW)], idxv)
    pltpu.async_copy(ew_hbm.at[idxv.at[pl.ds(0, 128)]], rows_a, sem).wait()
    pltpu.async_copy(ew_hbm.at[idxv.at[pl.ds(128, 128)]], rows_b, sem).wait()
    pltpu.sync_copy(rows_a, zq_hbm.at[pl.ds(base, 128)])
    pltpu.sync_copy(rows_b, zq_hbm.at[pl.ds(base + 128, 128)])

    def zero_body(i, carry):
        hist[pl.ds(i * 16, 16)] = jnp.zeros((16,), jnp.float32)
        return carry

    lax.fori_loop(0, _NUM_EMB // 16, zero_body, 0)

    ones = jnp.ones((16,), jnp.float32)

    def hist_body(i, carry):
        v = idxv[pl.ds(i * 16, 16)]
        plsc.addupdate_scatter(hist, [v], ones)
        return carry

    lax.fori_loop(0, _BW // 16, hist_body, 0)
    pltpu.sync_copy(hist, hist_hbm.at[wid])


@functools.cache
def _get_sc_gather():
    return functools.partial(
        pl.kernel,
        out_type=[
            jax.ShapeDtypeStruct((_NTOK, _EMB_DIM), jnp.float32),
            jax.ShapeDtypeStruct((_NW, _NUM_EMB), jnp.float32),
        ],
        mesh=plsc.VectorSubcoreMesh(core_axis_name="c", subcore_axis_name="s"),
        compiler_params=pltpu.CompilerParams(needs_layout_passes=False),
        scratch_types=[
            pltpu.VMEM((_BW,), jnp.int32),
            pltpu.VMEM((128, _EMB_DIM), jnp.float32),
            pltpu.VMEM((128, _EMB_DIM), jnp.float32),
            pltpu.VMEM((_NUM_EMB,), jnp.float32),
            pltpu.SemaphoreType.DMA,
        ],
    )(_sc_gather_body)


def _final_body(hist_ref, dsum_ref, loss_ref, perp_ref):
    counts = jnp.sum(hist_ref[...], axis=0, keepdims=True)
    avg = counts * (1.0 / float(_NTOK))
    plogp = avg * jnp.log(jnp.maximum(avg, 1e-10))
    perp_ref[0, 0] = jnp.exp(-jnp.sum(plogp))
    loss_ref[0, 0] = (1.0 + _COMMIT) * dsum_ref[0, 0] / float(_NTOK * _EMB_DIM)


def kernel(z_e, emb_w):
    b, c, h, w = z_e.shape
    zf = jnp.transpose(z_e.astype(jnp.float32), (0, 2, 3, 1)).reshape(_NTOK, _EMB_DIM)
    ew = emb_w.astype(jnp.float32)

    std = pl.pallas_call(
        _stats_body,
        out_shape=jax.ShapeDtypeStruct((1, _EMB_DIM), jnp.float32),
    )(zf)

    nblk = _NTOK // _TM
    idx3, dsum = pl.pallas_call(
        _argmin_body,
        grid=(nblk,),
        in_specs=[
            pl.BlockSpec((_TM, _EMB_DIM), lambda t: (t, 0)),
            pl.BlockSpec((1, _EMB_DIM), lambda t: (0, 0)),
            pl.BlockSpec((_NUM_EMB, _EMB_DIM), lambda t: (0, 0)),
        ],
        out_specs=[
            pl.BlockSpec((1, 1, _TM), lambda t: (t, 0, 0)),
            pl.BlockSpec(memory_space=pltpu.SMEM),
        ],
        out_shape=[
            jax.ShapeDtypeStruct((nblk, 1, _TM), jnp.int32),
            jax.ShapeDtypeStruct((1, 1), jnp.float32),
        ],
    )(zf, std, ew)
    idx = idx3.reshape(_NTOK)

    zq, hist = _get_sc_gather()(ew, idx)

    loss, perp = pl.pallas_call(
        _final_body,
        in_specs=[
            pl.BlockSpec((_NW, _NUM_EMB), lambda: (0, 0)),
            pl.BlockSpec(memory_space=pltpu.SMEM),
        ],
        out_specs=[
            pl.BlockSpec(memory_space=pltpu.SMEM),
            pl.BlockSpec(memory_space=pltpu.SMEM),
        ],
        out_shape=[
            jax.ShapeDtypeStruct((1, 1), jnp.float32),
            jax.ShapeDtypeStruct((1, 1), jnp.float32),
        ],
    )(hist, dsum)

    z_q_st = jnp.transpose(zq.reshape(b, h, w, c), (0, 3, 1, 2))
    return (z_q_st, loss.reshape(()), perp.reshape(()), idx.reshape(b, h, w))


# speedup vs baseline: 1.0682x; 1.0682x over previous
import functools

import jax
import jax.numpy as jnp
from jax import lax
from jax.experimental import pallas as pl
from jax.experimental.pallas import tpu as pltpu
from jax.experimental.pallas import tpu_sc as plsc

_NUM_EMB = 8192
_EMB_DIM = 256
_COMMIT = 0.25
_EPS = 1e-05
_NTOK = 8192

_TM = 2048
_CK = 4096

_NW = 32
_BW = _NTOK // _NW


def _stats_body(zf_ref, std_ref):
    zf = zf_ref[...]
    mu = jnp.sum(zf, axis=0, keepdims=True) / float(_NTOK)
    dev = zf - mu
    var = jnp.sum(dev * dev, axis=0, keepdims=True) / float(_NTOK - 1)
    std_ref[...] = 1.0 / jnp.maximum(jnp.sqrt(var), _EPS)


def _argmin_body(zf_ref, std_ref, ew_ref, idx_ref, dsum_ref):
    t = pl.program_id(0)
    zn = zf_ref[...] * std_ref[...]
    z2 = jnp.sum(zn * zn, axis=1, keepdims=True)
    znb = zn.astype(jnp.bfloat16)
    best = jnp.full((_TM,), jnp.inf, dtype=jnp.float32)
    besti = jnp.zeros((_TM,), dtype=jnp.int32)
    for c in range(_NUM_EMB // _CK):
        ew_c = ew_ref[pl.ds(c * _CK, _CK), :]
        m = lax.dot_general(znb, ew_c.astype(jnp.bfloat16),
                            (((1,), (1,)), ((), ())),
                            preferred_element_type=jnp.float32)
        e2 = jnp.sum(ew_c * ew_c, axis=1)
        d = (z2 - 2.0 * m) + e2[None, :]
        lm = jnp.min(d, axis=1)
        la = jnp.argmin(d, axis=1).astype(jnp.int32) + c * _CK
        upd = lm < best
        besti = jnp.where(upd, la, besti)
        best = jnp.where(upd, lm, best)
    idx_ref[0, 0, :] = besti

    @pl.when(t == 0)
    def _():
        dsum_ref[0, 0] = 0.0

    dsum_ref[0, 0] += jnp.sum(best)


def _sc_gather_body(ew_hbm, idx_hbm, zq_hbm, hist_hbm,
                    idxv, rows_a, rows_b, hist, sem):
    wid = lax.axis_index("s") * 2 + lax.axis_index("c")
    base = wid * _BW
    pltpu.sync_copy(idx_hbm.at[pl.ds(base, _BW)], idxv)
    pltpu.async_copy(ew_hbm.at[idxv.at[pl.ds(0, 128)]], rows_a, sem).wait()
    pltpu.async_copy(ew_hbm.at[idxv.at[pl.ds(128, 128)]], rows_b, sem).wait()
    pltpu.sync_copy(rows_a, zq_hbm.at[pl.ds(base, 128)])
    pltpu.sync_copy(rows_b, zq_hbm.at[pl.ds(base + 128, 128)])

    def zero_body(i, carry):
        hist[pl.ds(i * 16, 16)] = jnp.zeros((16,), jnp.float32)
        return carry

    lax.fori_loop(0, _NUM_EMB // 16, zero_body, 0)

    ones = jnp.ones((16,), jnp.float32)

    def hist_body(i, carry):
        v = idxv[pl.ds(i * 16, 16)]
        plsc.addupdate_scatter(hist, [v], ones)
        return carry

    lax.fori_loop(0, _BW // 16, hist_body, 0)
    pltpu.sync_copy(hist, hist_hbm.at[wid])


@functools.cache
def _get_sc_gather():
    return functools.partial(
        pl.kernel,
        out_type=[
            jax.ShapeDtypeStruct((_NTOK, _EMB_DIM), jnp.float32),
            jax.ShapeDtypeStruct((_NW, _NUM_EMB), jnp.float32),
        ],
        mesh=plsc.VectorSubcoreMesh(core_axis_name="c", subcore_axis_name="s"),
        compiler_params=pltpu.CompilerParams(needs_layout_passes=False),
        scratch_types=[
            pltpu.VMEM((_BW,), jnp.int32),
            pltpu.VMEM((128, _EMB_DIM), jnp.float32),
            pltpu.VMEM((128, _EMB_DIM), jnp.float32),
            pltpu.VMEM((_NUM_EMB,), jnp.float32),
            pltpu.SemaphoreType.DMA,
        ],
    )(_sc_gather_body)


def _final_body(hist_ref, dsum_ref, loss_ref, perp_ref):
    counts = jnp.sum(hist_ref[...], axis=0, keepdims=True)
    avg = counts * (1.0 / float(_NTOK))
    plogp = avg * jnp.log(jnp.maximum(avg, 1e-10))
    perp_ref[0, 0] = jnp.exp(-jnp.sum(plogp))
    loss_ref[0, 0] = (1.0 + _COMMIT) * dsum_ref[0, 0] / float(_NTOK * _EMB_DIM)


def kernel(z_e, emb_w):
    b, c, h, w = z_e.shape
    zf = jnp.transpose(z_e.astype(jnp.float32), (0, 2, 3, 1)).reshape(_NTOK, _EMB_DIM)
    ew = emb_w.astype(jnp.float32)

    std = pl.pallas_call(
        _stats_body,
        out_shape=jax.ShapeDtypeStruct((1, _EMB_DIM), jnp.float32),
    )(zf)

    nblk = _NTOK // _TM
    idx3, dsum = pl.pallas_call(
        _argmin_body,
        grid=(nblk,),
        in_specs=[
            pl.BlockSpec((_TM, _EMB_DIM), lambda t: (t, 0)),
            pl.BlockSpec((1, _EMB_DIM), lambda t: (0, 0)),
            pl.BlockSpec((_NUM_EMB, _EMB_DIM), lambda t: (0, 0)),
        ],
        out_specs=[
            pl.BlockSpec((1, 1, _TM), lambda t: (t, 0, 0)),
            pl.BlockSpec(memory_space=pltpu.SMEM),
        ],
        out_shape=[
            jax.ShapeDtypeStruct((nblk, 1, _TM), jnp.int32),
            jax.ShapeDtypeStruct((1, 1), jnp.float32),
        ],
    )(zf, std, ew)
    idx = idx3.reshape(_NTOK)

    zq, hist = _get_sc_gather()(ew, idx)

    loss, perp = pl.pallas_call(
        _final_body,
        in_specs=[
            pl.BlockSpec((_NW, _NUM_EMB), lambda: (0, 0)),
            pl.BlockSpec(memory_space=pltpu.SMEM),
        ],
        out_specs=[
            pl.BlockSpec(memory_space=pltpu.SMEM),
            pl.BlockSpec(memory_space=pltpu.SMEM),
        ],
        out_shape=[
            jax.ShapeDtypeStruct((1, 1), jnp.float32),
            jax.ShapeDtypeStruct((1, 1), jnp.float32),
        ],
    )(hist, dsum)

    z_q_st = jnp.transpose(zq.reshape(b, h, w, c), (0, 3, 1, 2))
    return (z_q_st, loss.reshape(()), perp.reshape(()), idx.reshape(b, h, w))
